# baseline (speedup 1.0000x reference)
import functools

import numpy as np
import jax
import jax.numpy as jnp
from jax import lax
from jax.experimental import pallas as pl
from jax.experimental.pallas import tpu as pltpu
from jax.experimental.pallas import tpu_sc as plsc

_B = 16384
_D = 32
_NC = 2
_NS = 16
_NW = _NC * _NS
_BPW = _B // _NW
_CHUNK = 128
_NCHUNK = _BPW // _CHUNK
_L = 16
_NAGE = 11
_NCOMB = 3 * _NAGE

_BOUNDS = tuple(float(x) for x in np.linspace(0.0, 100.0, num=10))

_mesh = plsc.VectorSubcoreMesh(core_axis_name="c", subcore_axis_name="s")


@functools.partial(
    pl.kernel,
    out_type=jax.ShapeDtypeStruct((_B, 128), jnp.float32),
    mesh=_mesh,
    compiler_params=pltpu.CompilerParams(use_tc_tiling_on_sc=False),
    scratch_types=[
        pltpu.VMEM((_BPW, 128), jnp.float32),
        pltpu.VMEM((_NCHUNK, _CHUNK), jnp.int32),
        pltpu.VMEM((_BPW,), jnp.int32),
        pltpu.VMEM((_BPW,), jnp.int32),
        pltpu.VMEM((_BPW,), jnp.float32),
        pltpu.VMEM((_NCHUNK, _CHUNK), jnp.int32),
        pltpu.VMEM((_BPW, _D), jnp.float32),
        pltpu.VMEM((_BPW, 2 * _D), jnp.float32),
        pltpu.VMEM_SHARED((_NCOMB, 2 * _D), jnp.float32),
        pltpu.SemaphoreType.DMA,
        pltpu.SemaphoreType.DMA,
        pltpu.SemaphoreType.DMA,
    ],
)
def _sc_lookup(uid_hbm, gid_hbm, age_hbm, ut2_hbm, ct_hbm, out_hbm,
               ubig_v, qidx_v, uid_v, gid_v, age_v, cidx_v, urows_v, crows_v,
               ct_sh, sem, csem, osem):
    sid = lax.axis_index("s")
    wid = sid * _NC + lax.axis_index("c")
    base = wid * _BPW

    @pl.when(sid == 0)
    def _():
        pltpu.sync_copy(ct_hbm, ct_sh)

    stage_u = pltpu.async_copy(uid_hbm.at[pl.ds(base, _BPW)], uid_v, sem)
    stage_g = pltpu.async_copy(gid_hbm.at[pl.ds(base, _BPW)], gid_v, sem)
    stage_a = pltpu.async_copy(age_hbm.at[pl.ds(base, _BPW)], age_v, sem)
    stage_u.wait()

    two = jnp.full((_L,), 2, jnp.int32)
    def qcomp(i, carry):
        u = uid_v[pl.ds(i * _L, _L)]
        qidx_v[i // (_CHUNK // _L), pl.ds((i % (_CHUNK // _L)) * _L, _L)] = (
            lax.shift_right_logical(u, two))
        return carry
    lax.fori_loop(0, _BPW // _L, qcomp, 0)

    gathers = []
    for c in range(_NCHUNK):
        gathers.append(pltpu.async_copy(
            ut2_hbm.at[qidx_v.at[c]],
            ubig_v.at[pl.ds(c * _CHUNK, _CHUNK)], sem))
    stage_g.wait()
    stage_a.wait()

    eleven = jnp.full((_L,), _NAGE, jnp.int32)
    one = jnp.ones((_L,), jnp.int32)
    zero = jnp.zeros((_L,), jnp.int32)
    for c in range(_NCHUNK):
        def bkt(i, carry, c=c):
            a = age_v[pl.ds(c * _CHUNK + i * _L, _L)]
            g = gid_v[pl.ds(c * _CHUNK + i * _L, _L)]
            b = g * eleven
            for t in _BOUNDS:
                tv = jnp.full((_L,), t, jnp.float32)
                b = b + jnp.where(a >= tv, one, zero)
            cidx_v[c, pl.ds(i * _L, _L)] = b
            return carry
        lax.fori_loop(0, _CHUNK // _L, bkt, 0)

    plsc.subcore_barrier()

    comb_gathers = []
    for c in range(_NCHUNK):
        comb_gathers.append(pltpu.async_copy(
            ct_sh.at[cidx_v.at[c]], crows_v.at[pl.ds(c * _CHUNK, _CHUNK)], csem))

    for h in gathers:
        h.wait()
    wr_u = pltpu.async_copy(ubig_v, out_hbm.at[pl.ds(base, _BPW)], osem)
    for h in comb_gathers:
        h.wait()
    wr_u.wait()


_RB = 2048


def _tc_transpose_body(x_ref, o_ref):
    x = x_ref[...]
    lane = lax.broadcasted_iota(jnp.int32, (_RB, 128), 1)
    x = jnp.where(lane < 96, x, 0.0)
    ek = lax.broadcasted_iota(jnp.int32, (128, 96), 0)
    ec = lax.broadcasted_iota(jnp.int32, (128, 96), 1)
    eye = (ek == ec).astype(jnp.float32)
    o_ref[...] = lax.dot_general(
        eye, x, (((0,), (1,)), ((), ())),
        preferred_element_type=jnp.float32,
        precision=lax.Precision.HIGHEST,
    )


_tc_transpose = pl.pallas_call(
    _tc_transpose_body,
    out_shape=jax.ShapeDtypeStruct((96, _B), jnp.float32),
    grid=(_B // _RB,),
    in_specs=[pl.BlockSpec((_RB, 128), lambda i: (i, 0))],
    out_specs=pl.BlockSpec((96, _RB), lambda i: (0, i)),
)


@jax.jit
def kernel(user_id, gender, age, user_table, gender_table, age_table):
    comb = jnp.concatenate(
        [jnp.repeat(gender_table, _NAGE, axis=0), jnp.tile(age_table, (3, 1))],
        axis=1,
    )
    ut2 = jnp.pad(user_table, ((0, 7), (0, 0))).reshape(25002, 128)
    staged = _sc_lookup(user_id, gender, age, ut2, comb)
    out_t = _tc_transpose(staged)
    return out_t.T


# speedup vs baseline: 1.3499x; 1.3499x over previous
import functools

import numpy as np
import jax
import jax.numpy as jnp
from jax import lax
from jax.experimental import pallas as pl
from jax.experimental.pallas import tpu as pltpu
from jax.experimental.pallas import tpu_sc as plsc

_B = 16384
_D = 32
_NC = 2
_NS = 16
_NW = _NC * _NS
_BPW = _B // _NW
_CHUNK = 128
_NCHUNK = _BPW // _CHUNK
_L = 16
_NAGE = 11
_NCOMB = 3 * _NAGE

_BOUNDS = tuple(float(x) for x in np.linspace(0.0, 100.0, num=10))

_mesh = plsc.VectorSubcoreMesh(core_axis_name="c", subcore_axis_name="s")


@functools.partial(
    pl.kernel,
    out_type=jax.ShapeDtypeStruct((_B, 128), jnp.float32),
    mesh=_mesh,
    compiler_params=pltpu.CompilerParams(use_tc_tiling_on_sc=False),
    scratch_types=[
        pltpu.VMEM((_BPW,), jnp.int32),
        pltpu.VMEM((_BPW,), jnp.int32),
        pltpu.VMEM((_BPW,), jnp.float32),
        pltpu.VMEM((_NCHUNK, _CHUNK), jnp.int32),
        pltpu.VMEM((_BPW, 128), jnp.float32),
        pltpu.VMEM((_BPW, 2 * _D), jnp.float32),
        pltpu.VMEM_SHARED((_NCOMB, 2 * _D), jnp.float32),
        pltpu.SemaphoreType.DMA,
        pltpu.SemaphoreType.DMA,
        pltpu.SemaphoreType.DMA,
    ],
)
def _sc_lookup(uid_hbm, gid_hbm, age_hbm, ut_hbm, ct_hbm, out_hbm,
               uid_v, gid_v, age_v, cidx_v, ubig_v, crows_v, ct_sh,
               sem, csem, osem):
    sid = lax.axis_index("s")
    wid = sid * _NC + lax.axis_index("c")
    base = wid * _BPW

    @pl.when(sid == 0)
    def _():
        pltpu.sync_copy(ct_hbm, ct_sh)

    stage_u = pltpu.async_copy(uid_hbm.at[pl.ds(base, _BPW)], uid_v, sem)
    stage_g = pltpu.async_copy(gid_hbm.at[pl.ds(base, _BPW)], gid_v, sem)
    stage_a = pltpu.async_copy(age_hbm.at[pl.ds(base, _BPW)], age_v, sem)
    stage_u.wait()

    gathers = []
    for c in range(_NCHUNK):
        gathers.append(pltpu.async_copy(
            ut_hbm.at[uid_v.at[pl.ds(c * _CHUNK, _CHUNK)]],
            ubig_v.at[pl.ds(c * _CHUNK, _CHUNK)], sem))
    stage_g.wait()
    stage_a.wait()

    eleven = jnp.full((_L,), _NAGE, jnp.int32)
    one = jnp.ones((_L,), jnp.int32)
    zero = jnp.zeros((_L,), jnp.int32)
    for c in range(_NCHUNK):
        def bkt(i, carry, c=c):
            a = age_v[pl.ds(c * _CHUNK + i * _L, _L)]
            g = gid_v[pl.ds(c * _CHUNK + i * _L, _L)]
            b = g * eleven
            for t in _BOUNDS:
                tv = jnp.full((_L,), t, jnp.float32)
                b = b + jnp.where(a >= tv, one, zero)
            cidx_v[c, pl.ds(i * _L, _L)] = b
            return carry
        lax.fori_loop(0, _CHUNK // _L, bkt, 0)

    plsc.subcore_barrier()

    comb_gathers = []
    for c in range(_NCHUNK):
        comb_gathers.append(pltpu.async_copy(
            ct_sh.at[cidx_v.at[c]], crows_v.at[pl.ds(c * _CHUNK, _CHUNK)], csem))

    for h in gathers:
        h.wait()
    wr_u = pltpu.async_copy(ubig_v.at[:, pl.ds(0, _D)],
                            out_hbm.at[pl.ds(base, _BPW), pl.ds(0, _D)], osem)
    for h in comb_gathers:
        h.wait()
    wr_c = pltpu.async_copy(crows_v, out_hbm.at[pl.ds(base, _BPW), pl.ds(_D, 2 * _D)], osem)
    wr_u.wait()
    wr_c.wait()


_PKB = 8192


def _tc_pack_body(x_ref, o_ref):
    x = x_ref[...]
    ek = lax.broadcasted_iota(jnp.int32, (_D, _D), 0)
    ec = lax.broadcasted_iota(jnp.int32, (_D, _D), 1)
    eye = (ek == ec).astype(jnp.float32)
    y = lax.dot_general(
        x, eye, (((0,), (0,)), ((), ())),
        preferred_element_type=jnp.float32,
        precision=lax.Precision.HIGHEST,
    )
    o_ref[:, :, 0:_D] = y.reshape(_PKB // 8, 8, _D)


_tc_pack = pl.pallas_call(
    _tc_pack_body,
    out_shape=jax.ShapeDtypeStruct((12501, 8, 128), jnp.float32),
    grid=(13,),
    in_specs=[pl.BlockSpec((_D, _PKB), lambda i: (0, i))],
    out_specs=pl.BlockSpec((_PKB // 8, 8, 128), lambda i: (i, 0, 0)),
)


_RB = 2048


def _tc_transpose_body(x_ref, o_ref):
    x = x_ref[...]
    lane = lax.broadcasted_iota(jnp.int32, (_RB, 128), 1)
    x = jnp.where(lane < 96, x, 0.0)
    ek = lax.broadcasted_iota(jnp.int32, (128, 96), 0)
    ec = lax.broadcasted_iota(jnp.int32, (128, 96), 1)
    eye = (ek == ec).astype(jnp.float32)
    o_ref[...] = lax.dot_general(
        eye, x, (((0,), (1,)), ((), ())),
        preferred_element_type=jnp.float32,
        precision=lax.Precision.HIGHEST,
    )


_tc_transpose = pl.pallas_call(
    _tc_transpose_body,
    out_shape=jax.ShapeDtypeStruct((96, _B), jnp.float32),
    grid=(_B // _RB,),
    in_specs=[pl.BlockSpec((_RB, 128), lambda i: (i, 0))],
    out_specs=pl.BlockSpec((96, _RB), lambda i: (0, i)),
)


@jax.jit
def kernel(user_id, gender, age, user_table, gender_table, age_table):
    comb = jnp.concatenate(
        [jnp.repeat(gender_table, _NAGE, axis=0), jnp.tile(age_table, (3, 1))],
        axis=1,
    )
    ut3 = _tc_pack(user_table.T).reshape(100008, 128)
    staged = _sc_lookup(user_id, gender, age, ut3, comb)
    out_t = _tc_transpose(staged)
    return out_t.T


# speedup vs baseline: 1.4055x; 1.0412x over previous
import functools

import numpy as np
import jax
import jax.numpy as jnp
from jax import lax
from jax.experimental import pallas as pl
from jax.experimental.pallas import tpu as pltpu
from jax.experimental.pallas import tpu_sc as plsc

_B = 16384
_D = 32
_NC = 2
_NS = 16
_NW = _NC * _NS
_BPW = _B // _NW
_CHUNK = 128
_NCHUNK = _BPW // _CHUNK
_L = 16
_NAGE = 11
_NCOMB = 3 * _NAGE

_BOUNDS = tuple(float(x) for x in np.linspace(0.0, 100.0, num=10))

_mesh = plsc.VectorSubcoreMesh(core_axis_name="c", subcore_axis_name="s")


@functools.partial(
    pl.kernel,
    out_type=jax.ShapeDtypeStruct((_B, 128), jnp.float32),
    mesh=_mesh,
    compiler_params=pltpu.CompilerParams(use_tc_tiling_on_sc=False),
    scratch_types=[
        pltpu.VMEM((_BPW,), jnp.int32),
        pltpu.VMEM((_BPW,), jnp.int32),
        pltpu.VMEM((_BPW,), jnp.float32),
        pltpu.VMEM((_NCHUNK, _CHUNK), jnp.int32),
        pltpu.VMEM((_BPW, _D), jnp.float32),
        pltpu.VMEM((_BPW, 2 * _D), jnp.float32),
        pltpu.VMEM_SHARED((_NCOMB, 2 * _D), jnp.float32),
        pltpu.SemaphoreType.DMA,
        pltpu.SemaphoreType.DMA,
        pltpu.SemaphoreType.DMA,
    ],
)
def _sc_lookup(uid_hbm, gid_hbm, age_hbm, ut_hbm, ct_hbm, out_hbm,
               uid_v, gid_v, age_v, cidx_v, urows_v, crows_v, ct_sh,
               sem, csem, osem):
    sid = lax.axis_index("s")
    wid = sid * _NC + lax.axis_index("c")
    base = wid * _BPW

    @pl.when(sid == 0)
    def _():
        pltpu.sync_copy(ct_hbm, ct_sh)

    stage_u = pltpu.async_copy(uid_hbm.at[pl.ds(base, _BPW)], uid_v, sem)
    stage_g = pltpu.async_copy(gid_hbm.at[pl.ds(base, _BPW)], gid_v, sem)
    stage_a = pltpu.async_copy(age_hbm.at[pl.ds(base, _BPW)], age_v, sem)
    stage_u.wait()

    gathers = []
    for c in range(_NCHUNK):
        gathers.append(pltpu.async_copy(
            ut_hbm.at[uid_v.at[pl.ds(c * _CHUNK, _CHUNK)]],
            urows_v.at[pl.ds(c * _CHUNK, _CHUNK)], sem))
    stage_g.wait()
    stage_a.wait()

    eleven = jnp.full((_L,), _NAGE, jnp.int32)
    one = jnp.ones((_L,), jnp.int32)
    zero = jnp.zeros((_L,), jnp.int32)
    for c in range(_NCHUNK):
        def bkt(i, carry, c=c):
            a = age_v[pl.ds(c * _CHUNK + i * _L, _L)]
            g = gid_v[pl.ds(c * _CHUNK + i * _L, _L)]
            b = g * eleven
            for t in _BOUNDS:
                tv = jnp.full((_L,), t, jnp.float32)
                b = b + jnp.where(a >= tv, one, zero)
            cidx_v[c, pl.ds(i * _L, _L)] = b
            return carry
        lax.fori_loop(0, _CHUNK // _L, bkt, 0)

    plsc.subcore_barrier()

    comb_gathers = []
    for c in range(_NCHUNK):
        comb_gathers.append(pltpu.async_copy(
            ct_sh.at[cidx_v.at[c]], crows_v.at[pl.ds(c * _CHUNK, _CHUNK)], csem))

    for h in gathers:
        h.wait()
    wr_u = pltpu.async_copy(urows_v, out_hbm.at[pl.ds(base, _BPW), pl.ds(0, _D)], osem)
    for h in comb_gathers:
        h.wait()
    wr_c = pltpu.async_copy(crows_v, out_hbm.at[pl.ds(base, _BPW), pl.ds(_D, 2 * _D)], osem)
    wr_u.wait()
    wr_c.wait()


_RB = 2048


def _tc_transpose_body(x_ref, o_ref):
    x = x_ref[...]
    lane = lax.broadcasted_iota(jnp.int32, (_RB, 128), 1)
    x = jnp.where(lane < 96, x, 0.0)
    ek = lax.broadcasted_iota(jnp.int32, (128, 96), 0)
    ec = lax.broadcasted_iota(jnp.int32, (128, 96), 1)
    eye = (ek == ec).astype(jnp.float32)
    o_ref[...] = lax.dot_general(
        eye, x, (((0,), (1,)), ((), ())),
        preferred_element_type=jnp.float32,
        precision=lax.Precision.HIGHEST,
    )


_tc_transpose = pl.pallas_call(
    _tc_transpose_body,
    out_shape=jax.ShapeDtypeStruct((96, _B), jnp.float32),
    grid=(_B // _RB,),
    in_specs=[pl.BlockSpec((_RB, 128), lambda i: (i, 0))],
    out_specs=pl.BlockSpec((96, _RB), lambda i: (0, i)),
)


@jax.jit
def kernel(user_id, gender, age, user_table, gender_table, age_table):
    comb = jnp.concatenate(
        [jnp.repeat(gender_table, _NAGE, axis=0), jnp.tile(age_table, (3, 1))],
        axis=1,
    )
    staged = _sc_lookup(user_id, gender, age, user_table, comb)
    out_t = _tc_transpose(staged)
    return out_t.T
